# Initial kernel scaffold; baseline (speedup 1.0000x reference)
import jax
import jax.numpy as jnp
from jax.experimental import pallas as pl


def kernel(x, edge_index, wpe, gat_W, att_src, att_dst, gat_b, ln2_scale, ln2_bias, fc_W, fc_b, proj_W, proj_b, lnf_scale, lnf_bias):
    raise NotImplementedError("write your pallas kernel here")


# speedup vs baseline: 39.9993x; 39.9993x over previous
import jax
import jax.numpy as jnp
from jax.experimental import pallas as pl

_B, _N, _D, _H, _C, _L, _FF = 4, 1024, 768, 12, 64, 6, 3072
_E = 65536
_BN = _B * _N
_PREC = jax.lax.Precision.HIGHEST

_EB = 1024
_NEB = _E // _EB
_RB = 256
_RA = 256
_RM = 256
_NA = _N // _RA


def _cnt_body(dst_ref, src_ref, out_ref):
    step = pl.program_id(0)
    dst = dst_ref[0, 0, :]
    src = src_ref[0, 0, :]
    ohT_dst = (dst[None, :] == jax.lax.broadcasted_iota(jnp.int32, (_N, _EB), 0)
               ).astype(jnp.bfloat16)
    oh_src = (src[:, None] == jax.lax.broadcasted_iota(jnp.int32, (_EB, _N), 1)
              ).astype(jnp.bfloat16)
    part = jax.lax.dot_general(ohT_dst, oh_src, (((1,), (0,)), ((), ())),
                               preferred_element_type=jnp.float32)

    @pl.when(step == 0)
    def _():
        out_ref[...] = part

    @pl.when(step > 0)
    def _():
        out_ref[...] = out_ref[...] + part


def _build_cnt(edge_index):
    dst = edge_index[1].reshape(_NEB, 1, _EB)
    src = edge_index[0].reshape(_NEB, 1, _EB)
    return pl.pallas_call(
        _cnt_body,
        grid=(_NEB,),
        in_specs=[pl.BlockSpec((1, 1, _EB), lambda e: (e, 0, 0)),
                  pl.BlockSpec((1, 1, _EB), lambda e: (e, 0, 0))],
        out_specs=pl.BlockSpec((_N, _N), lambda e: (0, 0)),
        out_shape=jax.ShapeDtypeStruct((_N, _N), jnp.float32),
    )(dst, src)


def _xp_body(h_ref, w_ref, m_ref, xp_ref, asad_ref):
    xp = jnp.dot(h_ref[...], w_ref[...],
                 preferred_element_type=jnp.float32, precision=_PREC)
    xp_ref[...] = xp
    asad_ref[...] = jnp.dot(xp, m_ref[...],
                            preferred_element_type=jnp.float32, precision=_PREC)


def _embed_xp_body(x_ref, pe_ref, w_ref, m_ref, h_ref, xp_ref, asad_ref):
    h = x_ref[...] + pe_ref[...]
    h_ref[...] = h
    xp = jnp.dot(h, w_ref[...],
                 preferred_element_type=jnp.float32, precision=_PREC)
    xp_ref[...] = xp
    asad_ref[...] = jnp.dot(xp, m_ref[...],
                            preferred_element_type=jnp.float32, precision=_PREC)


def _call_embed(x_flat, wpe, w, m):
    g = _BN // _RB
    return pl.pallas_call(
        _embed_xp_body,
        grid=(g,),
        in_specs=[
            pl.BlockSpec((_RB, _D), lambda i: (i, 0)),
            pl.BlockSpec((_RB, _D), lambda i: (i % (_N // _RB), 0)),
            pl.BlockSpec((_D, _D), lambda i: (0, 0)),
            pl.BlockSpec((_D, 2 * _H), lambda i: (0, 0)),
        ],
        out_specs=[
            pl.BlockSpec((_RB, _D), lambda i: (i, 0)),
            pl.BlockSpec((_RB, _D), lambda i: (i, 0)),
            pl.BlockSpec((_RB, 2 * _H), lambda i: (i, 0)),
        ],
        out_shape=[
            jax.ShapeDtypeStruct((_BN, _D), jnp.float32),
            jax.ShapeDtypeStruct((_BN, _D), jnp.float32),
            jax.ShapeDtypeStruct((_BN, 2 * _H), jnp.float32),
        ],
    )(x_flat, wpe, w, m)


def _call_xp(h, w, m):
    g = _BN // _RB
    return pl.pallas_call(
        _xp_body,
        grid=(g,),
        in_specs=[
            pl.BlockSpec((_RB, _D), lambda i: (i, 0)),
            pl.BlockSpec((_D, _D), lambda i: (0, 0)),
            pl.BlockSpec((_D, 2 * _H), lambda i: (0, 0)),
        ],
        out_specs=[
            pl.BlockSpec((_RB, _D), lambda i: (i, 0)),
            pl.BlockSpec((_RB, 2 * _H), lambda i: (i, 0)),
        ],
        out_shape=[
            jax.ShapeDtypeStruct((_BN, _D), jnp.float32),
            jax.ShapeDtypeStruct((_BN, 2 * _H), jnp.float32),
        ],
    )(h, w, m)


def _attn_body(cnt_ref, xp_ref, asf_ref, asb_ref, res_ref, gatb_ref,
               lns_ref, lnb_ref, h1_ref, hn_ref):
    cnt = cnt_ref[...]
    cols = []
    for h in range(_H):
        a_src = asf_ref[0, :, h]
        a_dst = asb_ref[0, :, _H + h]
        logit = a_dst[:, None] + a_src[None, :]
        lr = jnp.where(logit >= 0, logit, 0.2 * logit)
        m = jnp.max(lr, axis=1, keepdims=True)
        a = cnt * jnp.exp(lr - m)
        den = jnp.sum(a, axis=1, keepdims=True)
        xp_h = xp_ref[0, :, h * _C:(h + 1) * _C]
        o = jax.lax.dot_general(a, xp_h, (((1,), (0,)), ((), ())),
                                preferred_element_type=jnp.float32,
                                precision=_PREC)
        cols.append(o / (den + 1e-16))
    h1 = jnp.concatenate(cols, axis=1) + gatb_ref[...] + res_ref[0]
    h1_ref[0] = h1
    mu = jnp.mean(h1, axis=1, keepdims=True)
    var = jnp.mean((h1 - mu) ** 2, axis=1, keepdims=True)
    hn_ref[0] = (h1 - mu) * jax.lax.rsqrt(var + 1e-5) * lns_ref[...] + lnb_ref[...]


def _call_attn(cnt, xp, asad, res, gatb, lns, lnb):
    return pl.pallas_call(
        _attn_body,
        grid=(_B, _NA),
        in_specs=[
            pl.BlockSpec((_RA, _N), lambda b, i: (i, 0)),
            pl.BlockSpec((1, _N, _D), lambda b, i: (b, 0, 0)),
            pl.BlockSpec((1, _N, 2 * _H), lambda b, i: (b, 0, 0)),
            pl.BlockSpec((1, _RA, 2 * _H), lambda b, i: (b, i, 0)),
            pl.BlockSpec((1, _RA, _D), lambda b, i: (b, i, 0)),
            pl.BlockSpec((1, _D), lambda b, i: (0, 0)),
            pl.BlockSpec((1, _D), lambda b, i: (0, 0)),
            pl.BlockSpec((1, _D), lambda b, i: (0, 0)),
        ],
        out_specs=[
            pl.BlockSpec((1, _RA, _D), lambda b, i: (b, i, 0)),
            pl.BlockSpec((1, _RA, _D), lambda b, i: (b, i, 0)),
        ],
        out_shape=[
            jax.ShapeDtypeStruct((_B, _N, _D), jnp.float32),
            jax.ShapeDtypeStruct((_B, _N, _D), jnp.float32),
        ],
    )(cnt, xp, asad, asad, res, gatb, lns, lnb)


def _fc_body(hn_ref, w_ref, b_ref, out_ref):
    y = jnp.dot(hn_ref[...], w_ref[...],
                preferred_element_type=jnp.float32, precision=_PREC) + b_ref[...]
    k = 0.7978845608028654
    out_ref[...] = 0.5 * y * (1.0 + jnp.tanh(k * (y + 0.044715 * (y * y * y))))


def _call_fc(hn, w, b):
    g = _BN // _RM
    return pl.pallas_call(
        _fc_body,
        grid=(g,),
        in_specs=[
            pl.BlockSpec((_RM, _D), lambda i: (i, 0)),
            pl.BlockSpec((_D, _FF), lambda i: (0, 0)),
            pl.BlockSpec((1, _FF), lambda i: (0, 0)),
        ],
        out_specs=pl.BlockSpec((_RM, _FF), lambda i: (i, 0)),
        out_shape=jax.ShapeDtypeStruct((_BN, _FF), jnp.float32),
    )(hn, w, b)


def _proj_body(ff_ref, w_ref, b_ref, res_ref, out_ref):
    out_ref[...] = (jnp.dot(ff_ref[...], w_ref[...],
                            preferred_element_type=jnp.float32, precision=_PREC)
                    + b_ref[...] + res_ref[...])


def _call_proj(ff, w, b, res):
    g = _BN // _RM
    return pl.pallas_call(
        _proj_body,
        grid=(g,),
        in_specs=[
            pl.BlockSpec((_RM, _FF), lambda i: (i, 0)),
            pl.BlockSpec((_FF, _D), lambda i: (0, 0)),
            pl.BlockSpec((1, _D), lambda i: (0, 0)),
            pl.BlockSpec((_RM, _D), lambda i: (i, 0)),
        ],
        out_specs=pl.BlockSpec((_RM, _D), lambda i: (i, 0)),
        out_shape=jax.ShapeDtypeStruct((_BN, _D), jnp.float32),
    )(ff, w, b, res)


def _lnf_body(h_ref, s_ref, b_ref, out_ref):
    xx = h_ref[...]
    mu = jnp.mean(xx, axis=1, keepdims=True)
    var = jnp.mean((xx - mu) ** 2, axis=1, keepdims=True)
    out_ref[...] = (xx - mu) * jax.lax.rsqrt(var + 1e-5) * s_ref[...] + b_ref[...]


def _call_lnf(h, s, b):
    g = _BN // _RM
    return pl.pallas_call(
        _lnf_body,
        grid=(g,),
        in_specs=[
            pl.BlockSpec((_RM, _D), lambda i: (i, 0)),
            pl.BlockSpec((1, _D), lambda i: (0, 0)),
            pl.BlockSpec((1, _D), lambda i: (0, 0)),
        ],
        out_specs=pl.BlockSpec((_RM, _D), lambda i: (i, 0)),
        out_shape=jax.ShapeDtypeStruct((_BN, _D), jnp.float32),
    )(h, s, b)


def kernel(x, edge_index, wpe, gat_W, att_src, att_dst, gat_b, ln2_scale,
           ln2_bias, fc_W, fc_b, proj_W, proj_b, lnf_scale, lnf_bias):
    cnt = _build_cnt(edge_index)

    eye_h = jnp.eye(_H, dtype=jnp.float32)
    m_src = (att_src[:, :, :, None] * eye_h[None, :, None, :]).reshape(_L, _D, _H)
    m_dst = (att_dst[:, :, :, None] * eye_h[None, :, None, :]).reshape(_L, _D, _H)
    m_all = jnp.concatenate([m_src, m_dst], axis=2)

    x_flat = x.reshape(_BN, _D)
    h = None
    for l in range(_L):
        if l == 0:
            h, xp, asad = _call_embed(x_flat, wpe, gat_W[0], m_all[0])
        else:
            xp, asad = _call_xp(h, gat_W[l], m_all[l])
        h1, hn = _call_attn(cnt,
                            xp.reshape(_B, _N, _D),
                            asad.reshape(_B, _N, 2 * _H),
                            h.reshape(_B, _N, _D),
                            gat_b[l][None], ln2_scale[l][None], ln2_bias[l][None])
        ff = _call_fc(hn.reshape(_BN, _D), fc_W[l], fc_b[l][None])
        h = _call_proj(ff, proj_W[l], proj_b[l][None], h1.reshape(_BN, _D))
    out = _call_lnf(h, lnf_scale[None], lnf_bias[None])
    return out.reshape(_B, _N, _D)


# speedup vs baseline: 78.1908x; 1.9548x over previous
import jax
import jax.numpy as jnp
from jax.experimental import pallas as pl

_B, _N, _D, _H, _C, _L, _FF = 4, 1024, 768, 12, 64, 6, 3072
_E = 65536
_BN = _B * _N

_EB = 1024
_NEB = _E // _EB
_RB = 256
_RA = 256
_NA = _N // _RA


def _cnt_body(dst_ref, src_ref, out_ref):
    step = pl.program_id(0)
    dst = dst_ref[0, 0, :]
    src = src_ref[0, 0, :]
    ohT_dst = (dst[None, :] == jax.lax.broadcasted_iota(jnp.int32, (_N, _EB), 0)
               ).astype(jnp.bfloat16)
    oh_src = (src[:, None] == jax.lax.broadcasted_iota(jnp.int32, (_EB, _N), 1)
              ).astype(jnp.bfloat16)
    part = jax.lax.dot_general(ohT_dst, oh_src, (((1,), (0,)), ((), ())),
                               preferred_element_type=jnp.float32)

    @pl.when(step == 0)
    def _():
        out_ref[...] = part

    @pl.when(step > 0)
    def _():
        out_ref[...] = out_ref[...] + part


def _build_cnt(edge_index):
    dst = edge_index[1].reshape(_NEB, 1, _EB)
    src = edge_index[0].reshape(_NEB, 1, _EB)
    return pl.pallas_call(
        _cnt_body,
        grid=(_NEB,),
        in_specs=[pl.BlockSpec((1, 1, _EB), lambda e: (e, 0, 0)),
                  pl.BlockSpec((1, 1, _EB), lambda e: (e, 0, 0))],
        out_specs=pl.BlockSpec((_N, _N), lambda e: (0, 0)),
        out_shape=jax.ShapeDtypeStruct((_N, _N), jnp.float32),
    )(dst, src)


def _embed_xp_body(x_ref, pe_ref, w_ref, m_ref, h_ref, xp_ref, asad_ref):
    h = x_ref[...] + pe_ref[...]
    h_ref[...] = h
    xp = jnp.dot(h, w_ref[...], preferred_element_type=jnp.float32)
    xp_ref[...] = xp
    asad_ref[...] = jnp.dot(xp, m_ref[...], preferred_element_type=jnp.float32)


def _call_embed(x_flat, wpe, w, m):
    g = _BN // _RB
    return pl.pallas_call(
        _embed_xp_body,
        grid=(g,),
        in_specs=[
            pl.BlockSpec((_RB, _D), lambda i: (i, 0)),
            pl.BlockSpec((_RB, _D), lambda i: (i % (_N // _RB), 0)),
            pl.BlockSpec((_D, _D), lambda i: (0, 0)),
            pl.BlockSpec((_D, 2 * _H), lambda i: (0, 0)),
        ],
        out_specs=[
            pl.BlockSpec((_RB, _D), lambda i: (i, 0)),
            pl.BlockSpec((_RB, _D), lambda i: (i, 0)),
            pl.BlockSpec((_RB, 2 * _H), lambda i: (i, 0)),
        ],
        out_shape=[
            jax.ShapeDtypeStruct((_BN, _D), jnp.float32),
            jax.ShapeDtypeStruct((_BN, _D), jnp.float32),
            jax.ShapeDtypeStruct((_BN, 2 * _H), jnp.float32),
        ],
    )(x_flat, wpe, w, m)


def _attn_body(cnt_ref, xp_ref, asf_ref, asb_ref, res_ref, gatb_ref,
               lns_ref, lnb_ref, h1_ref, hn_ref):
    cnt = cnt_ref[...]
    cols = []
    for h in range(_H):
        as_full = asf_ref[0, :, h]
        ad = asb_ref[0, :, _H + h]
        mx = jnp.max(as_full)
        s = ad + mx
        m = jnp.where(s >= 0, s, 0.2 * s)
        q1 = jnp.exp(as_full)
        q2 = jnp.exp(0.2 * as_full)
        p1 = jnp.exp(ad - m)
        p2 = jnp.exp(0.2 * ad - m)
        t = jnp.maximum(p1[:, None] * q1[None, :], p2[:, None] * q2[None, :])
        a = cnt * t
        den = jnp.sum(a, axis=1, keepdims=True)
        xp_h = xp_ref[0, :, h * _C:(h + 1) * _C]
        o = jax.lax.dot_general(a, xp_h, (((1,), (0,)), ((), ())),
                                preferred_element_type=jnp.float32)
        cols.append(o / (den + 1e-16))
    h1 = jnp.concatenate(cols, axis=1) + gatb_ref[...] + res_ref[0]
    h1_ref[0] = h1
    mu = jnp.mean(h1, axis=1, keepdims=True)
    var = jnp.mean((h1 - mu) ** 2, axis=1, keepdims=True)
    hn_ref[0] = (h1 - mu) * jax.lax.rsqrt(var + 1e-5) * lns_ref[...] + lnb_ref[...]


def _call_attn(cnt, xp, asad, res, gatb, lns, lnb):
    return pl.pallas_call(
        _attn_body,
        grid=(_B, _NA),
        in_specs=[
            pl.BlockSpec((_RA, _N), lambda b, i: (i, 0)),
            pl.BlockSpec((1, _N, _D), lambda b, i: (b, 0, 0)),
            pl.BlockSpec((1, _N, 2 * _H), lambda b, i: (b, 0, 0)),
            pl.BlockSpec((1, _RA, 2 * _H), lambda b, i: (b, i, 0)),
            pl.BlockSpec((1, _RA, _D), lambda b, i: (b, i, 0)),
            pl.BlockSpec((1, _D), lambda b, i: (0, 0)),
            pl.BlockSpec((1, _D), lambda b, i: (0, 0)),
            pl.BlockSpec((1, _D), lambda b, i: (0, 0)),
        ],
        out_specs=[
            pl.BlockSpec((1, _RA, _D), lambda b, i: (b, i, 0)),
            pl.BlockSpec((1, _RA, _D), lambda b, i: (b, i, 0)),
        ],
        out_shape=[
            jax.ShapeDtypeStruct((_B, _N, _D), jnp.float32),
            jax.ShapeDtypeStruct((_B, _N, _D), jnp.float32),
        ],
    )(cnt, xp, asad, asad, res, gatb, lns, lnb)


def _gelu(y):
    k = 0.7978845608028654
    return 0.5 * y * (1.0 + jnp.tanh(k * (y + 0.044715 * (y * y * y))))


def _mlp_mid_body(hn_ref, res_ref, fcw_ref, fcb_ref, pjw_ref, pjb_ref,
                  w_ref, m_ref, h_ref, xp_ref, asad_ref):
    y = _gelu(jnp.dot(hn_ref[...], fcw_ref[...],
                      preferred_element_type=jnp.float32) + fcb_ref[...])
    h = (jnp.dot(y, pjw_ref[...], preferred_element_type=jnp.float32)
         + pjb_ref[...] + res_ref[...])
    h_ref[...] = h
    xp = jnp.dot(h, w_ref[...], preferred_element_type=jnp.float32)
    xp_ref[...] = xp
    asad_ref[...] = jnp.dot(xp, m_ref[...], preferred_element_type=jnp.float32)


def _call_mlp_mid(hn, res, fcw, fcb, pjw, pjb, w, m):
    g = _BN // _RB
    return pl.pallas_call(
        _mlp_mid_body,
        grid=(g,),
        in_specs=[
            pl.BlockSpec((_RB, _D), lambda i: (i, 0)),
            pl.BlockSpec((_RB, _D), lambda i: (i, 0)),
            pl.BlockSpec((_D, _FF), lambda i: (0, 0)),
            pl.BlockSpec((1, _FF), lambda i: (0, 0)),
            pl.BlockSpec((_FF, _D), lambda i: (0, 0)),
            pl.BlockSpec((1, _D), lambda i: (0, 0)),
            pl.BlockSpec((_D, _D), lambda i: (0, 0)),
            pl.BlockSpec((_D, 2 * _H), lambda i: (0, 0)),
        ],
        out_specs=[
            pl.BlockSpec((_RB, _D), lambda i: (i, 0)),
            pl.BlockSpec((_RB, _D), lambda i: (i, 0)),
            pl.BlockSpec((_RB, 2 * _H), lambda i: (i, 0)),
        ],
        out_shape=[
            jax.ShapeDtypeStruct((_BN, _D), jnp.float32),
            jax.ShapeDtypeStruct((_BN, _D), jnp.float32),
            jax.ShapeDtypeStruct((_BN, 2 * _H), jnp.float32),
        ],
    )(hn, res, fcw, fcb, pjw, pjb, w, m)


def _mlp_last_body(hn_ref, res_ref, fcw_ref, fcb_ref, pjw_ref, pjb_ref,
                   lnfs_ref, lnfb_ref, out_ref):
    y = _gelu(jnp.dot(hn_ref[...], fcw_ref[...],
                      preferred_element_type=jnp.float32) + fcb_ref[...])
    h = (jnp.dot(y, pjw_ref[...], preferred_element_type=jnp.float32)
         + pjb_ref[...] + res_ref[...])
    mu = jnp.mean(h, axis=1, keepdims=True)
    var = jnp.mean((h - mu) ** 2, axis=1, keepdims=True)
    out_ref[...] = ((h - mu) * jax.lax.rsqrt(var + 1e-5) * lnfs_ref[...]
                    + lnfb_ref[...])


def _call_mlp_last(hn, res, fcw, fcb, pjw, pjb, lnfs, lnfb):
    g = _BN // _RB
    return pl.pallas_call(
        _mlp_last_body,
        grid=(g,),
        in_specs=[
            pl.BlockSpec((_RB, _D), lambda i: (i, 0)),
            pl.BlockSpec((_RB, _D), lambda i: (i, 0)),
            pl.BlockSpec((_D, _FF), lambda i: (0, 0)),
            pl.BlockSpec((1, _FF), lambda i: (0, 0)),
            pl.BlockSpec((_FF, _D), lambda i: (0, 0)),
            pl.BlockSpec((1, _D), lambda i: (0, 0)),
            pl.BlockSpec((1, _D), lambda i: (0, 0)),
            pl.BlockSpec((1, _D), lambda i: (0, 0)),
        ],
        out_specs=pl.BlockSpec((_RB, _D), lambda i: (i, 0)),
        out_shape=jax.ShapeDtypeStruct((_BN, _D), jnp.float32),
    )(hn, res, fcw, fcb, pjw, pjb, lnfs, lnfb)


def kernel(x, edge_index, wpe, gat_W, att_src, att_dst, gat_b, ln2_scale,
           ln2_bias, fc_W, fc_b, proj_W, proj_b, lnf_scale, lnf_bias):
    cnt = _build_cnt(edge_index)

    eye_h = jnp.eye(_H, dtype=jnp.float32)
    m_src = (att_src[:, :, :, None] * eye_h[None, :, None, :]).reshape(_L, _D, _H)
    m_dst = (att_dst[:, :, :, None] * eye_h[None, :, None, :]).reshape(_L, _D, _H)
    m_all = jnp.concatenate([m_src, m_dst], axis=2)

    h, xp, asad = _call_embed(x.reshape(_BN, _D), wpe, gat_W[0], m_all[0])
    out = None
    for l in range(_L):
        h1, hn = _call_attn(cnt,
                            xp.reshape(_B, _N, _D),
                            asad.reshape(_B, _N, 2 * _H),
                            h.reshape(_B, _N, _D),
                            gat_b[l][None], ln2_scale[l][None], ln2_bias[l][None])
        hn_f = hn.reshape(_BN, _D)
        h1_f = h1.reshape(_BN, _D)
        if l < _L - 1:
            h, xp, asad = _call_mlp_mid(hn_f, h1_f, fc_W[l], fc_b[l][None],
                                        proj_W[l], proj_b[l][None],
                                        gat_W[l + 1], m_all[l + 1])
        else:
            out = _call_mlp_last(hn_f, h1_f, fc_W[l], fc_b[l][None],
                                 proj_W[l], proj_b[l][None],
                                 lnf_scale[None], lnf_bias[None])
    return out.reshape(_B, _N, _D)


# speedup vs baseline: 85.2851x; 1.0907x over previous
import functools

import jax
import jax.numpy as jnp
from jax import lax
from jax.experimental import pallas as pl
from jax.experimental.pallas import tpu as pltpu
from jax.experimental.pallas import tpu_sc as plsc

_B, _N, _D, _H, _C, _L, _FF = 4, 1024, 768, 12, 64, 6, 3072
_E = 65536
_BN = _B * _N

_EB = 1024
_NEB = _E // _EB
_RB = 256
_RA = 256
_NA = _N // _RA


def _cnt_body(dst_ref, src_ref, out_ref):
    step = pl.program_id(0)
    dst = dst_ref[0, 0, :]
    src = src_ref[0, 0, :]
    ohT_dst = (dst[None, :] == jax.lax.broadcasted_iota(jnp.int32, (_N, _EB), 0)
               ).astype(jnp.bfloat16)
    oh_src = (src[:, None] == jax.lax.broadcasted_iota(jnp.int32, (_EB, _N), 1)
              ).astype(jnp.bfloat16)
    part = jax.lax.dot_general(ohT_dst, oh_src, (((1,), (0,)), ((), ())),
                               preferred_element_type=jnp.float32)

    @pl.when(step == 0)
    def _():
        out_ref[...] = part

    @pl.when(step > 0)
    def _():
        out_ref[...] = out_ref[...] + part


def _build_cnt(edge_index):
    dst = edge_index[1].reshape(_NEB, 1, _EB)
    src = edge_index[0].reshape(_NEB, 1, _EB)
    return pl.pallas_call(
        _cnt_body,
        grid=(_NEB,),
        in_specs=[pl.BlockSpec((1, 1, _EB), lambda e: (e, 0, 0)),
                  pl.BlockSpec((1, 1, _EB), lambda e: (e, 0, 0))],
        out_specs=pl.BlockSpec((_N, _N), lambda e: (0, 0)),
        out_shape=jax.ShapeDtypeStruct((_N, _N), jnp.float32),
    )(dst, src)


_EPW = _E // 16
_ZCH = 16384


def _cnt_sc_body(dst_hbm, src_hbm, out_hbm, dst_v, src_v, idx2_v, ones_v,
                 zero_v, acc_sh):
    c = lax.axis_index("c")
    s = lax.axis_index("s")

    def _fill(k, _):
        zero_v[pl.ds(k * 16, 16)] = jnp.zeros((16,), jnp.float32)
        return 0

    lax.fori_loop(0, _ZCH // 16, _fill, 0)

    def _fill1(k, _):
        ones_v[pl.ds(k * 16, 16)] = jnp.ones((16,), jnp.float32)
        return 0

    lax.fori_loop(0, 8, _fill1, 0)

    @pl.when(c == 0)
    def _():
        for k in range(_N * _N // 16 // _ZCH):
            pltpu.sync_copy(zero_v,
                            acc_sh.at[pl.ds(s * (_N * _N // 16) + k * _ZCH,
                                            _ZCH)])
        pltpu.sync_copy(dst_hbm.at[pl.ds(s * _EPW, _EPW)], dst_v)
        pltpu.sync_copy(src_hbm.at[pl.ds(s * _EPW, _EPW)], src_v)
        for j in range(_EPW // 128):
            def _flat(k, _):
                off = j * 128 + k * 16
                d = dst_v[pl.ds(off, 16)]
                sr = src_v[pl.ds(off, 16)]
                idx2_v[j, pl.ds(k * 16, 16)] = d * _N + sr
                return 0

            lax.fori_loop(0, 8, _flat, 0)

    plsc.subcore_barrier()

    @pl.when(c == 0)
    def _():
        for j in range(_EPW // 128):
            pltpu.sync_copy(ones_v, acc_sh.at[idx2_v.at[j]], add=True)

    plsc.subcore_barrier()

    @pl.when(c == 0)
    def _():
        sl = _N * _N // 16
        pltpu.sync_copy(acc_sh.at[pl.ds(s * sl, sl)],
                        out_hbm.at[pl.ds(s * sl, sl)])


def _build_cnt_sc(edge_index):
    mesh = plsc.VectorSubcoreMesh(core_axis_name="c", subcore_axis_name="s")
    k = functools.partial(
        pl.kernel,
        mesh=mesh,
        out_type=jax.ShapeDtypeStruct((_N * _N,), jnp.float32),
        scratch_types=[
            pltpu.VMEM((_EPW,), jnp.int32),
            pltpu.VMEM((_EPW,), jnp.int32),
            pltpu.VMEM((_EPW // 128, 128), jnp.int32),
            pltpu.VMEM((128,), jnp.float32),
            pltpu.VMEM((_ZCH,), jnp.float32),
            pltpu.VMEM_SHARED((_N * _N,), jnp.float32),
        ],
    )(_cnt_sc_body)
    return k(edge_index[1], edge_index[0]).reshape(_N, _N)


def _embed_xp_body(x_ref, pe_ref, w_ref, m_ref, h_ref, xp_ref, asad_ref):
    h = x_ref[...] + pe_ref[...]
    h_ref[...] = h
    xp = jnp.dot(h, w_ref[...], preferred_element_type=jnp.float32)
    xp_ref[...] = xp
    asad_ref[...] = jnp.dot(xp, m_ref[...], preferred_element_type=jnp.float32)


def _call_embed(x_flat, wpe, w, m):
    g = _BN // _RB
    return pl.pallas_call(
        _embed_xp_body,
        grid=(g,),
        in_specs=[
            pl.BlockSpec((_RB, _D), lambda i: (i, 0)),
            pl.BlockSpec((_RB, _D), lambda i: (i % (_N // _RB), 0)),
            pl.BlockSpec((_D, _D), lambda i: (0, 0)),
            pl.BlockSpec((_D, 2 * _H), lambda i: (0, 0)),
        ],
        out_specs=[
            pl.BlockSpec((_RB, _D), lambda i: (i, 0)),
            pl.BlockSpec((_RB, _D), lambda i: (i, 0)),
            pl.BlockSpec((_RB, 2 * _H), lambda i: (i, 0)),
        ],
        out_shape=[
            jax.ShapeDtypeStruct((_BN, _D), jnp.float32),
            jax.ShapeDtypeStruct((_BN, _D), jnp.float32),
            jax.ShapeDtypeStruct((_BN, 2 * _H), jnp.float32),
        ],
    )(x_flat, wpe, w, m)


def _attn_body(cnt_ref, xp_ref, asf_ref, asb_ref, res_ref, gatb_ref,
               lns_ref, lnb_ref, h1_ref, hn_ref):
    cnt = cnt_ref[...]
    cols = []
    for h in range(_H):
        as_full = asf_ref[0, :, h]
        ad = asb_ref[0, :, _H + h]
        mx = jnp.max(as_full)
        s = ad + mx
        m = jnp.where(s >= 0, s, 0.2 * s)
        q1 = jnp.exp(as_full)
        q2 = jnp.exp(0.2 * as_full)
        p1 = jnp.exp(ad - m)
        p2 = jnp.exp(0.2 * ad - m)
        t = jnp.maximum(p1[:, None] * q1[None, :], p2[:, None] * q2[None, :])
        a = cnt * t
        den = jnp.sum(a, axis=1, keepdims=True)
        xp_h = xp_ref[0, :, h * _C:(h + 1) * _C]
        o = jax.lax.dot_general(a, xp_h, (((1,), (0,)), ((), ())),
                                preferred_element_type=jnp.float32)
        cols.append(o / (den + 1e-16))
    h1 = jnp.concatenate(cols, axis=1) + gatb_ref[...] + res_ref[0]
    h1_ref[0] = h1
    mu = jnp.mean(h1, axis=1, keepdims=True)
    var = jnp.mean((h1 - mu) ** 2, axis=1, keepdims=True)
    hn_ref[0] = (h1 - mu) * jax.lax.rsqrt(var + 1e-5) * lns_ref[...] + lnb_ref[...]


def _call_attn(cnt, xp, asad, res, gatb, lns, lnb):
    return pl.pallas_call(
        _attn_body,
        grid=(_B, _NA),
        in_specs=[
            pl.BlockSpec((_RA, _N), lambda b, i: (i, 0)),
            pl.BlockSpec((1, _N, _D), lambda b, i: (b, 0, 0)),
            pl.BlockSpec((1, _N, 2 * _H), lambda b, i: (b, 0, 0)),
            pl.BlockSpec((1, _RA, 2 * _H), lambda b, i: (b, i, 0)),
            pl.BlockSpec((1, _RA, _D), lambda b, i: (b, i, 0)),
            pl.BlockSpec((1, _D), lambda b, i: (0, 0)),
            pl.BlockSpec((1, _D), lambda b, i: (0, 0)),
            pl.BlockSpec((1, _D), lambda b, i: (0, 0)),
        ],
        out_specs=[
            pl.BlockSpec((1, _RA, _D), lambda b, i: (b, i, 0)),
            pl.BlockSpec((1, _RA, _D), lambda b, i: (b, i, 0)),
        ],
        out_shape=[
            jax.ShapeDtypeStruct((_B, _N, _D), jnp.float32),
            jax.ShapeDtypeStruct((_B, _N, _D), jnp.float32),
        ],
    )(cnt, xp, asad, asad, res, gatb, lns, lnb)


def _gelu(y):
    k = 0.7978845608028654
    return 0.5 * y * (1.0 + jnp.tanh(k * (y + 0.044715 * (y * y * y))))


def _mlp_mid_body(hn_ref, res_ref, fcw_ref, fcb_ref, pjw_ref, pjb_ref,
                  w_ref, m_ref, h_ref, xp_ref, asad_ref):
    y = _gelu(jnp.dot(hn_ref[...], fcw_ref[...],
                      preferred_element_type=jnp.float32) + fcb_ref[...])
    h = (jnp.dot(y, pjw_ref[...], preferred_element_type=jnp.float32)
         + pjb_ref[...] + res_ref[...])
    h_ref[...] = h
    xp = jnp.dot(h, w_ref[...], preferred_element_type=jnp.float32)
    xp_ref[...] = xp
    asad_ref[...] = jnp.dot(xp, m_ref[...], preferred_element_type=jnp.float32)


def _call_mlp_mid(hn, res, fcw, fcb, pjw, pjb, w, m):
    g = _BN // _RB
    return pl.pallas_call(
        _mlp_mid_body,
        grid=(g,),
        in_specs=[
            pl.BlockSpec((_RB, _D), lambda i: (i, 0)),
            pl.BlockSpec((_RB, _D), lambda i: (i, 0)),
            pl.BlockSpec((_D, _FF), lambda i: (0, 0)),
            pl.BlockSpec((1, _FF), lambda i: (0, 0)),
            pl.BlockSpec((_FF, _D), lambda i: (0, 0)),
            pl.BlockSpec((1, _D), lambda i: (0, 0)),
            pl.BlockSpec((_D, _D), lambda i: (0, 0)),
            pl.BlockSpec((_D, 2 * _H), lambda i: (0, 0)),
        ],
        out_specs=[
            pl.BlockSpec((_RB, _D), lambda i: (i, 0)),
            pl.BlockSpec((_RB, _D), lambda i: (i, 0)),
            pl.BlockSpec((_RB, 2 * _H), lambda i: (i, 0)),
        ],
        out_shape=[
            jax.ShapeDtypeStruct((_BN, _D), jnp.float32),
            jax.ShapeDtypeStruct((_BN, _D), jnp.float32),
            jax.ShapeDtypeStruct((_BN, 2 * _H), jnp.float32),
        ],
    )(hn, res, fcw, fcb, pjw, pjb, w, m)


def _mlp_last_body(hn_ref, res_ref, fcw_ref, fcb_ref, pjw_ref, pjb_ref,
                   lnfs_ref, lnfb_ref, out_ref):
    y = _gelu(jnp.dot(hn_ref[...], fcw_ref[...],
                      preferred_element_type=jnp.float32) + fcb_ref[...])
    h = (jnp.dot(y, pjw_ref[...], preferred_element_type=jnp.float32)
         + pjb_ref[...] + res_ref[...])
    mu = jnp.mean(h, axis=1, keepdims=True)
    var = jnp.mean((h - mu) ** 2, axis=1, keepdims=True)
    out_ref[...] = ((h - mu) * jax.lax.rsqrt(var + 1e-5) * lnfs_ref[...]
                    + lnfb_ref[...])


def _call_mlp_last(hn, res, fcw, fcb, pjw, pjb, lnfs, lnfb):
    g = _BN // _RB
    return pl.pallas_call(
        _mlp_last_body,
        grid=(g,),
        in_specs=[
            pl.BlockSpec((_RB, _D), lambda i: (i, 0)),
            pl.BlockSpec((_RB, _D), lambda i: (i, 0)),
            pl.BlockSpec((_D, _FF), lambda i: (0, 0)),
            pl.BlockSpec((1, _FF), lambda i: (0, 0)),
            pl.BlockSpec((_FF, _D), lambda i: (0, 0)),
            pl.BlockSpec((1, _D), lambda i: (0, 0)),
            pl.BlockSpec((1, _D), lambda i: (0, 0)),
            pl.BlockSpec((1, _D), lambda i: (0, 0)),
        ],
        out_specs=pl.BlockSpec((_RB, _D), lambda i: (i, 0)),
        out_shape=jax.ShapeDtypeStruct((_BN, _D), jnp.float32),
    )(hn, res, fcw, fcb, pjw, pjb, lnfs, lnfb)


def kernel(x, edge_index, wpe, gat_W, att_src, att_dst, gat_b, ln2_scale,
           ln2_bias, fc_W, fc_b, proj_W, proj_b, lnf_scale, lnf_bias):
    cnt = _build_cnt_sc(edge_index)

    eye_h = jnp.eye(_H, dtype=jnp.float32)
    m_src = (att_src[:, :, :, None] * eye_h[None, :, None, :]).reshape(_L, _D, _H)
    m_dst = (att_dst[:, :, :, None] * eye_h[None, :, None, :]).reshape(_L, _D, _H)
    m_all = jnp.concatenate([m_src, m_dst], axis=2)

    h, xp, asad = _call_embed(x.reshape(_BN, _D), wpe, gat_W[0], m_all[0])
    out = None
    for l in range(_L):
        h1, hn = _call_attn(cnt,
                            xp.reshape(_B, _N, _D),
                            asad.reshape(_B, _N, 2 * _H),
                            h.reshape(_B, _N, _D),
                            gat_b[l][None], ln2_scale[l][None], ln2_bias[l][None])
        hn_f = hn.reshape(_BN, _D)
        h1_f = h1.reshape(_BN, _D)
        if l < _L - 1:
            h, xp, asad = _call_mlp_mid(hn_f, h1_f, fc_W[l], fc_b[l][None],
                                        proj_W[l], proj_b[l][None],
                                        gat_W[l + 1], m_all[l + 1])
        else:
            out = _call_mlp_last(hn_f, h1_f, fc_W[l], fc_b[l][None],
                                 proj_W[l], proj_b[l][None],
                                 lnf_scale[None], lnf_bias[None])
    return out.reshape(_B, _N, _D)
